# baseline (speedup 1.0000x reference)
import functools

import jax
import jax.numpy as jnp
from jax import lax
from jax.experimental import pallas as pl
from jax.experimental.pallas import tpu as pltpu
from jax.experimental.pallas import tpu_sc as plsc

D = 128
NC, NS = 2, 16
NW = NC * NS
B = 16384 * 64
CHUNK = 128
ROWS = B // CHUNK
ROWS_W = ROWS // NW
NBUF = 4
N_IT = ROWS_W // NBUF
N_IT2 = N_IT // 2

_mesh = plsc.VectorSubcoreMesh(
    core_axis_name="c", subcore_axis_name="s", num_cores=NC, num_subcores=NS
)


@functools.partial(
    pl.kernel,
    out_type=jax.ShapeDtypeStruct((B, D), jnp.float32),
    mesh=_mesh,
    scratch_types=[
        pltpu.VMEM((2, NBUF, CHUNK), jnp.int32),
        pltpu.VMEM((NBUF, CHUNK, D), jnp.float32),
        [pltpu.SemaphoreType.DMA] * NBUF,
        [pltpu.SemaphoreType.DMA] * NBUF,
        [pltpu.SemaphoreType.DMA] * 2,
    ],
)
def _gather_kernel(x_hbm, table_hbm, out_hbm, idx_v, rows_v, gsems, osems, isems):
    wid = lax.axis_index("s") * NC + lax.axis_index("c")
    row0 = wid * ROWS_W
    last_blk = ROWS_W - NBUF

    def prefetch(it, p):
        blk = jnp.minimum(it * NBUF, last_blk)
        pltpu.async_copy(x_hbm.at[pl.ds(row0 + blk, NBUF)], idx_v.at[p], isems[p])

    def iteration(it, p, *, skip_osem_wait=False):
        rep_off = (wid * 2 + p) * VOCAB
        g_row = row0 + it * NBUF
        pltpu.make_async_copy(
            x_hbm.at[pl.ds(g_row, NBUF)], idx_v.at[p], isems[p]
        ).wait()
        for b in range(NBUF):
            for kk in range(CHUNK // 16):
                sl = idx_v.at[p].at[b][pl.ds(kk * 16, 16)]
                idx_v.at[p].at[b][pl.ds(kk * 16, 16)] = sl + rep_off
        gdescs = []
        for b in range(NBUF):
            out_slc = out_hbm.at[pl.ds((g_row + b) * CHUNK, CHUNK)]
            if not skip_osem_wait:
                pltpu.make_async_copy(rows_v.at[b], out_slc, osems[b]).wait()
            gdescs.append(
                pltpu.async_copy(
                    table_hbm.at[idx_v.at[p].at[b]], rows_v.at[b], gsems[b]
                )
            )
        for b in range(NBUF):
            gdescs[b].wait()
            pltpu.async_copy(
                rows_v.at[b],
                out_hbm.at[pl.ds((g_row + b) * CHUNK, CHUNK)],
                osems[b],
            )
        prefetch(it + 2, p)

    prefetch(0, 0)
    prefetch(1, 1)
    iteration(0, 0, skip_osem_wait=True)
    iteration(1, 1)

    def body(j, carry):
        iteration(2 * j, 0)
        iteration(2 * j + 1, 1)
        return carry

    lax.fori_loop(1, N_IT2, body, 0)

    last = row0 + (N_IT - 1) * NBUF
    for b in range(NBUF):
        pltpu.make_async_copy(
            rows_v.at[b],
            out_hbm.at[pl.ds((last + b) * CHUNK, CHUNK)],
            osems[b],
        ).wait()
    for p in range(2):
        pltpu.make_async_copy(
            x_hbm.at[pl.ds(row0 + last_blk, NBUF)], idx_v.at[p], isems[p]
        ).wait()


REP = 64
VOCAB = 1000


def kernel(x, table):
    table_rep = jnp.tile(table, (REP, 1))
    out = _gather_kernel(x.reshape(ROWS, CHUNK), table_rep)
    return out.reshape(x.shape[0], x.shape[1], D)


# speedup vs baseline: 1.0147x; 1.0147x over previous
import functools

import jax
import jax.numpy as jnp
from jax import lax
from jax.experimental import pallas as pl
from jax.experimental.pallas import tpu as pltpu
from jax.experimental.pallas import tpu_sc as plsc

D = 128
NC, NS = 2, 16
NW = NC * NS
B = 16384 * 64
CHUNK = 128
ROWS = B // CHUNK
ROWS_W = ROWS // NW
NBUF = 4
N_IT = ROWS_W // NBUF
N_IT2 = N_IT // 2

_mesh = plsc.VectorSubcoreMesh(
    core_axis_name="c", subcore_axis_name="s", num_cores=NC, num_subcores=NS
)


@functools.partial(
    pl.kernel,
    out_type=jax.ShapeDtypeStruct((B, D), jnp.float32),
    mesh=_mesh,
    scratch_types=[
        pltpu.VMEM((2, NBUF, CHUNK), jnp.int32),
        pltpu.VMEM((NBUF, CHUNK, D), jnp.float32),
        [pltpu.SemaphoreType.DMA] * NBUF,
        [pltpu.SemaphoreType.DMA] * NBUF,
        [pltpu.SemaphoreType.DMA] * 2,
    ],
)
def _gather_kernel(x_hbm, table_hbm, out_hbm, idx_v, rows_v, gsems, osems, isems):
    wid = lax.axis_index("s") * NC + lax.axis_index("c")
    row0 = wid * ROWS_W
    rep_off = (wid % REP) * VOCAB
    last_blk = ROWS_W - NBUF

    def prefetch(it, p):
        blk = jnp.minimum(it * NBUF, last_blk)
        pltpu.async_copy(x_hbm.at[pl.ds(row0 + blk, NBUF)], idx_v.at[p], isems[p])

    def iteration(it, p, *, skip_osem_wait=False):
        g_row = row0 + it * NBUF
        pltpu.make_async_copy(
            x_hbm.at[pl.ds(g_row, NBUF)], idx_v.at[p], isems[p]
        ).wait()
        for b in range(NBUF):
            for kk in range(CHUNK // 16):
                sl = idx_v.at[p].at[b][pl.ds(kk * 16, 16)]
                idx_v.at[p].at[b][pl.ds(kk * 16, 16)] = sl + rep_off
        gdescs = []
        for b in range(NBUF):
            out_slc = out_hbm.at[pl.ds((g_row + b) * CHUNK, CHUNK)]
            if not skip_osem_wait:
                pltpu.make_async_copy(rows_v.at[b], out_slc, osems[b]).wait()
            gdescs.append(
                pltpu.async_copy(
                    table_hbm.at[idx_v.at[p].at[b]], rows_v.at[b], gsems[b]
                )
            )
        for b in range(NBUF):
            gdescs[b].wait()
            pltpu.async_copy(
                rows_v.at[b],
                out_hbm.at[pl.ds((g_row + b) * CHUNK, CHUNK)],
                osems[b],
            )
        prefetch(it + 2, p)

    prefetch(0, 0)
    prefetch(1, 1)
    iteration(0, 0, skip_osem_wait=True)
    iteration(1, 1)

    def body(j, carry):
        iteration(2 * j, 0)
        iteration(2 * j + 1, 1)
        return carry

    lax.fori_loop(1, N_IT2, body, 0)

    last = row0 + (N_IT - 1) * NBUF
    for b in range(NBUF):
        pltpu.make_async_copy(
            rows_v.at[b],
            out_hbm.at[pl.ds((last + b) * CHUNK, CHUNK)],
            osems[b],
        ).wait()
    for p in range(2):
        pltpu.make_async_copy(
            x_hbm.at[pl.ds(row0 + last_blk, NBUF)], idx_v.at[p], isems[p]
        ).wait()


REP = 32
VOCAB = 1000


def kernel(x, table):
    table_rep = jnp.tile(table, (REP, 1))
    out = _gather_kernel(x.reshape(ROWS, CHUNK), table_rep)
    return out.reshape(x.shape[0], x.shape[1], D)
